# Initial kernel scaffold; baseline (speedup 1.0000x reference)
import jax
import jax.numpy as jnp
from jax.experimental import pallas as pl


def kernel(x, edge_index, eps, W1, b1, W2, b2, gamma, beta, lin_W, lin_b):
    raise NotImplementedError("write your pallas kernel here")


# speedup vs baseline: 5.0060x; 5.0060x over previous
import functools

import jax
import jax.numpy as jnp
from jax import lax
from jax.experimental import pallas as pl
from jax.experimental.pallas import tpu as pltpu
from jax.experimental.pallas import tpu_sc as plsc

N = 10000
F = 128
H = 512
E = 160000


_CHUNK = 128
_NUM_CHUNKS = E // _CHUNK
_NC = 2
_NS = 16
_NW = _NC * _NS
_CHUNKS_PER_W = -(-_NUM_CHUNKS // _NW)
_ROWS_PER_S = 632
_N_PAD = _NS * _ROWS_PER_S


def _sc_agg(x, src, dst, zeros):
    mesh = plsc.VectorSubcoreMesh(core_axis_name="c", subcore_axis_name="s")

    @functools.partial(
        pl.kernel,
        mesh=mesh,
        out_type=jax.ShapeDtypeStruct((_NC, _N_PAD, F), jnp.float32),
        scratch_types=[
            pltpu.VMEM((_CHUNK,), jnp.int32),
            pltpu.VMEM((_CHUNK,), jnp.int32),
            pltpu.VMEM((_CHUNK, F), jnp.float32),
            pltpu.VMEM_SHARED((_N_PAD, F), jnp.float32),
            pltpu.SemaphoreType.DMA,
        ],
    )
    def k(x_hbm, src_hbm, dst_hbm, zeros_hbm, out_hbm,
          src_v, dst_v, rows_v, agg_sh, sem):
        cidx = lax.axis_index("c")
        sidx = lax.axis_index("s")
        wid = sidx * _NC + cidx

        pltpu.sync_copy(zeros_hbm.at[pl.ds(sidx * _ROWS_PER_S, _ROWS_PER_S)],
                        agg_sh.at[pl.ds(sidx * _ROWS_PER_S, _ROWS_PER_S)])
        plsc.subcore_barrier()

        def body(j, _):
            cid = wid + j * _NW

            @pl.when(cid < _NUM_CHUNKS)
            def _():
                base = cid * _CHUNK
                pltpu.sync_copy(src_hbm.at[pl.ds(base, _CHUNK)], src_v)
                pltpu.sync_copy(dst_hbm.at[pl.ds(base, _CHUNK)], dst_v)
                pltpu.async_copy(x_hbm.at[src_v], rows_v, sem).wait()
                pltpu.sync_copy(rows_v, agg_sh.at[dst_v], add=True)
            return ()

        lax.fori_loop(0, _CHUNKS_PER_W, body, (), unroll=False)
        plsc.subcore_barrier()

        pltpu.sync_copy(agg_sh.at[pl.ds(sidx * _ROWS_PER_S, _ROWS_PER_S)],
                        out_hbm.at[cidx, pl.ds(sidx * _ROWS_PER_S, _ROWS_PER_S)])

    return k(x, src, dst, zeros)


_G = 10
_BLK = N // _G


def _mlp_stats_body(eps_ref, x_ref, parts_ref, w1_ref, b1_ref, w2_ref, b2_ref,
                    h_ref, s_ref, q_ref):
    i = pl.program_id(0)
    agg = parts_ref[0] + parts_ref[1]
    h0 = x_ref[...] * (1.0 + eps_ref[0]) + agg
    h1 = jnp.maximum(
        jnp.dot(h0, w1_ref[...], preferred_element_type=jnp.float32) + b1_ref[...],
        0.0)
    h2 = jnp.dot(h1, w2_ref[...], preferred_element_type=jnp.float32) + b2_ref[...]
    h_ref[...] = h2

    @pl.when(i == 0)
    def _():
        s_ref[...] = jnp.zeros_like(s_ref)
        q_ref[...] = jnp.zeros_like(q_ref)

    s_ref[...] += jnp.sum(h2, axis=0, keepdims=True)
    q_ref[...] += jnp.sum(h2 * h2, axis=0, keepdims=True)


def _bn_lin_body(h_ref, s_ref, q_ref, gamma_ref, beta_ref, lw_ref, lb_ref,
                 o_ref):
    mean = s_ref[...] * (1.0 / N)
    var = q_ref[...] * (1.0 / N) - mean * mean
    inv = lax.rsqrt(var + 1e-5)
    scale = gamma_ref[...] * inv
    shift = beta_ref[...] - mean * scale
    h3 = jnp.maximum(h_ref[...] * scale + shift, 0.0)
    o_ref[...] = (jnp.dot(h3, lw_ref[...], preferred_element_type=jnp.float32)
                  + lb_ref[...])


def kernel(x, edge_index, eps, W1, b1, W2, b2, gamma, beta, lin_W, lin_b):
    src = edge_index[0]
    dst = edge_index[1]
    zeros = jnp.zeros((_N_PAD, F), jnp.float32)
    parts = _sc_agg(x, src, dst, zeros)[:, :N, :]

    eps1 = jnp.reshape(eps, (1,))
    h2, s, q = pl.pallas_call(
        _mlp_stats_body,
        grid=(_G,),
        in_specs=[
            pl.BlockSpec(memory_space=pltpu.SMEM),
            pl.BlockSpec((_BLK, F), lambda i: (i, 0)),
            pl.BlockSpec((_NC, _BLK, F), lambda i: (0, i, 0)),
            pl.BlockSpec((F, H), lambda i: (0, 0)),
            pl.BlockSpec((1, H), lambda i: (0, 0)),
            pl.BlockSpec((H, H), lambda i: (0, 0)),
            pl.BlockSpec((1, H), lambda i: (0, 0)),
        ],
        out_specs=[
            pl.BlockSpec((_BLK, H), lambda i: (i, 0)),
            pl.BlockSpec((1, H), lambda i: (0, 0)),
            pl.BlockSpec((1, H), lambda i: (0, 0)),
        ],
        out_shape=[
            jax.ShapeDtypeStruct((N, H), jnp.float32),
            jax.ShapeDtypeStruct((1, H), jnp.float32),
            jax.ShapeDtypeStruct((1, H), jnp.float32),
        ],
    )(eps1, x, parts, W1, b1.reshape(1, H), W2, b2.reshape(1, H))

    out = pl.pallas_call(
        _bn_lin_body,
        grid=(_G,),
        in_specs=[
            pl.BlockSpec((_BLK, H), lambda i: (i, 0)),
            pl.BlockSpec((1, H), lambda i: (0, 0)),
            pl.BlockSpec((1, H), lambda i: (0, 0)),
            pl.BlockSpec((1, H), lambda i: (0, 0)),
            pl.BlockSpec((1, H), lambda i: (0, 0)),
            pl.BlockSpec((H, H), lambda i: (0, 0)),
            pl.BlockSpec((1, H), lambda i: (0, 0)),
        ],
        out_specs=pl.BlockSpec((_BLK, H), lambda i: (i, 0)),
        out_shape=jax.ShapeDtypeStruct((N, H), jnp.float32),
    )(h2, s, q, gamma.reshape(1, H), beta.reshape(1, H), lin_W,
      lin_b.reshape(1, H))
    return out


# speedup vs baseline: 6.8471x; 1.3678x over previous
import functools

import jax
import jax.numpy as jnp
from jax import lax
from jax.experimental import pallas as pl
from jax.experimental.pallas import tpu as pltpu
from jax.experimental.pallas import tpu_sc as plsc

N = 10000
F = 128
H = 512
E = 160000


_CHUNK = 125
_NUM_CHUNKS = E // _CHUNK
_NC = 2
_NS = 16
_NW = _NC * _NS
_CHUNKS_PER_W = _NUM_CHUNKS // _NW
_ROWS_PER_S = 632
_N_PAD = _NS * _ROWS_PER_S


def _sc_agg(x, src, dst, zeros):
    mesh = plsc.VectorSubcoreMesh(core_axis_name="c", subcore_axis_name="s")

    @functools.partial(
        pl.kernel,
        mesh=mesh,
        out_type=jax.ShapeDtypeStruct((_NC, _N_PAD, F), jnp.float32),
        scratch_types=[
            pltpu.VMEM((_CHUNKS_PER_W, _CHUNK), jnp.int32),
            pltpu.VMEM((_CHUNKS_PER_W, _CHUNK), jnp.int32),
            pltpu.VMEM((_CHUNK, F), jnp.float32),
            pltpu.VMEM((_CHUNK, F), jnp.float32),
            pltpu.VMEM_SHARED((_N_PAD, F), jnp.float32),
            pltpu.SemaphoreType.DMA,
            pltpu.SemaphoreType.DMA,
        ],
    )
    def k(x_hbm, src_hbm, dst_hbm, zeros_hbm, out_hbm,
          src_v, dst_v, rows0, rows1, agg_sh, sem0, sem1):
        cidx = lax.axis_index("c")
        sidx = lax.axis_index("s")
        wid = sidx * _NC + cidx
        cbase = wid * _CHUNKS_PER_W

        pltpu.sync_copy(src_hbm.at[pl.ds(cbase, _CHUNKS_PER_W)], src_v)
        pltpu.sync_copy(dst_hbm.at[pl.ds(cbase, _CHUNKS_PER_W)], dst_v)

        pltpu.sync_copy(zeros_hbm.at[pl.ds(sidx * _ROWS_PER_S, _ROWS_PER_S)],
                        agg_sh.at[pl.ds(sidx * _ROWS_PER_S, _ROWS_PER_S)])
        plsc.subcore_barrier()

        pltpu.async_copy(x_hbm.at[src_v.at[0]], rows0, sem0)

        def body(t, _):
            j0 = 2 * t
            pltpu.make_async_copy(x_hbm.at[src_v.at[j0]], rows0, sem0).wait()
            pltpu.async_copy(x_hbm.at[src_v.at[j0 + 1]], rows1, sem1)
            pltpu.sync_copy(rows0, agg_sh.at[dst_v.at[j0]], add=True)
            pltpu.make_async_copy(x_hbm.at[src_v.at[j0 + 1]], rows1, sem1).wait()

            @pl.when(t < _CHUNKS_PER_W // 2 - 1)
            def _():
                pltpu.async_copy(x_hbm.at[src_v.at[j0 + 2]], rows0, sem0)

            pltpu.sync_copy(rows1, agg_sh.at[dst_v.at[j0 + 1]], add=True)
            return ()

        lax.fori_loop(0, _CHUNKS_PER_W // 2, body, (), unroll=False)
        plsc.subcore_barrier()

        pltpu.sync_copy(agg_sh.at[pl.ds(sidx * _ROWS_PER_S, _ROWS_PER_S)],
                        out_hbm.at[cidx, pl.ds(sidx * _ROWS_PER_S, _ROWS_PER_S)])

    return k(x, src, dst, zeros)


_G = 10
_BLK = N // _G


def _mlp_stats_body(eps_ref, x_ref, parts_ref, w1_ref, b1_ref, w2_ref, b2_ref,
                    h_ref, s_ref, q_ref):
    i = pl.program_id(0)
    agg = parts_ref[0] + parts_ref[1]
    h0 = x_ref[...] * (1.0 + eps_ref[0]) + agg
    h1 = jnp.maximum(
        jnp.dot(h0, w1_ref[...], preferred_element_type=jnp.float32) + b1_ref[...],
        0.0)
    h2 = jnp.dot(h1, w2_ref[...], preferred_element_type=jnp.float32) + b2_ref[...]
    h_ref[...] = h2

    @pl.when(i == 0)
    def _():
        s_ref[...] = jnp.zeros_like(s_ref)
        q_ref[...] = jnp.zeros_like(q_ref)

    s_ref[...] += jnp.sum(h2, axis=0, keepdims=True)
    q_ref[...] += jnp.sum(h2 * h2, axis=0, keepdims=True)


def _bn_lin_body(h_ref, s_ref, q_ref, gamma_ref, beta_ref, lw_ref, lb_ref,
                 o_ref):
    mean = s_ref[...] * (1.0 / N)
    var = q_ref[...] * (1.0 / N) - mean * mean
    inv = lax.rsqrt(var + 1e-5)
    scale = gamma_ref[...] * inv
    shift = beta_ref[...] - mean * scale
    h3 = jnp.maximum(h_ref[...] * scale + shift, 0.0)
    o_ref[...] = (jnp.dot(h3, lw_ref[...], preferred_element_type=jnp.float32)
                  + lb_ref[...])


def kernel(x, edge_index, eps, W1, b1, W2, b2, gamma, beta, lin_W, lin_b):
    src = edge_index[0].reshape(_NUM_CHUNKS, _CHUNK)
    dst = edge_index[1].reshape(_NUM_CHUNKS, _CHUNK)
    zeros = jnp.zeros((_N_PAD, F), jnp.float32)
    parts = _sc_agg(x, src, dst, zeros)[:, :N, :]

    eps1 = jnp.reshape(eps, (1,))
    h2, s, q = pl.pallas_call(
        _mlp_stats_body,
        grid=(_G,),
        in_specs=[
            pl.BlockSpec(memory_space=pltpu.SMEM),
            pl.BlockSpec((_BLK, F), lambda i: (i, 0)),
            pl.BlockSpec((_NC, _BLK, F), lambda i: (0, i, 0)),
            pl.BlockSpec((F, H), lambda i: (0, 0)),
            pl.BlockSpec((1, H), lambda i: (0, 0)),
            pl.BlockSpec((H, H), lambda i: (0, 0)),
            pl.BlockSpec((1, H), lambda i: (0, 0)),
        ],
        out_specs=[
            pl.BlockSpec((_BLK, H), lambda i: (i, 0)),
            pl.BlockSpec((1, H), lambda i: (0, 0)),
            pl.BlockSpec((1, H), lambda i: (0, 0)),
        ],
        out_shape=[
            jax.ShapeDtypeStruct((N, H), jnp.float32),
            jax.ShapeDtypeStruct((1, H), jnp.float32),
            jax.ShapeDtypeStruct((1, H), jnp.float32),
        ],
    )(eps1, x, parts, W1, b1.reshape(1, H), W2, b2.reshape(1, H))

    out = pl.pallas_call(
        _bn_lin_body,
        grid=(_G,),
        in_specs=[
            pl.BlockSpec((_BLK, H), lambda i: (i, 0)),
            pl.BlockSpec((1, H), lambda i: (0, 0)),
            pl.BlockSpec((1, H), lambda i: (0, 0)),
            pl.BlockSpec((1, H), lambda i: (0, 0)),
            pl.BlockSpec((1, H), lambda i: (0, 0)),
            pl.BlockSpec((H, H), lambda i: (0, 0)),
            pl.BlockSpec((1, H), lambda i: (0, 0)),
        ],
        out_specs=pl.BlockSpec((_BLK, H), lambda i: (i, 0)),
        out_shape=jax.ShapeDtypeStruct((N, H), jnp.float32),
    )(h2, s, q, gamma.reshape(1, H), beta.reshape(1, H), lin_W,
      lin_b.reshape(1, H))
    return out


# speedup vs baseline: 6.8592x; 1.0018x over previous
import functools

import jax
import jax.numpy as jnp
from jax import lax
from jax.experimental import pallas as pl
from jax.experimental.pallas import tpu as pltpu
from jax.experimental.pallas import tpu_sc as plsc

N = 10000
F = 128
H = 512
E = 160000


_CHUNK = 125
_NUM_CHUNKS = E // _CHUNK
_NC = 2
_NS = 16
_NW = _NC * _NS
_CHUNKS_PER_W = _NUM_CHUNKS // _NW
_ROWS_PER_S = 632
_N_PAD = _NS * _ROWS_PER_S


def _sc_agg(x, src, dst, zeros):
    mesh = plsc.VectorSubcoreMesh(core_axis_name="c", subcore_axis_name="s")

    @functools.partial(
        pl.kernel,
        mesh=mesh,
        out_type=jax.ShapeDtypeStruct((_NC, _N_PAD, F), jnp.float32),
        scratch_types=[
            pltpu.VMEM((_CHUNKS_PER_W, _CHUNK), jnp.int32),
            pltpu.VMEM((_CHUNKS_PER_W, _CHUNK), jnp.int32),
            pltpu.VMEM((_CHUNK, F), jnp.float32),
            pltpu.VMEM((_CHUNK, F), jnp.float32),
            pltpu.VMEM_SHARED((_N_PAD, F), jnp.float32),
            pltpu.SemaphoreType.DMA,
            pltpu.SemaphoreType.DMA,
        ],
    )
    def k(x_hbm, src_hbm, dst_hbm, zeros_hbm, out_hbm,
          src_v, dst_v, rows0, rows1, agg_sh, sem0, sem1):
        cidx = lax.axis_index("c")
        sidx = lax.axis_index("s")
        wid = sidx * _NC + cidx
        cbase = wid * _CHUNKS_PER_W

        pltpu.sync_copy(src_hbm.at[pl.ds(cbase, _CHUNKS_PER_W)], src_v)
        pltpu.sync_copy(dst_hbm.at[pl.ds(cbase, _CHUNKS_PER_W)], dst_v)

        pltpu.sync_copy(zeros_hbm.at[pl.ds(sidx * _ROWS_PER_S, _ROWS_PER_S)],
                        agg_sh.at[pl.ds(sidx * _ROWS_PER_S, _ROWS_PER_S)])
        plsc.subcore_barrier()

        pltpu.async_copy(x_hbm.at[src_v.at[0]], rows0, sem0)

        def body(t, _):
            j0 = 2 * t
            pltpu.make_async_copy(x_hbm.at[src_v.at[j0]], rows0, sem0).wait()
            pltpu.async_copy(x_hbm.at[src_v.at[j0 + 1]], rows1, sem1)
            pltpu.sync_copy(rows0, agg_sh.at[dst_v.at[j0]], add=True)
            pltpu.make_async_copy(x_hbm.at[src_v.at[j0 + 1]], rows1, sem1).wait()

            @pl.when(t < _CHUNKS_PER_W // 2 - 1)
            def _():
                pltpu.async_copy(x_hbm.at[src_v.at[j0 + 2]], rows0, sem0)

            pltpu.sync_copy(rows1, agg_sh.at[dst_v.at[j0 + 1]], add=True)
            return ()

        lax.fori_loop(0, _CHUNKS_PER_W // 2, body, (), unroll=False)
        plsc.subcore_barrier()

        pltpu.sync_copy(agg_sh.at[pl.ds(sidx * _ROWS_PER_S, _ROWS_PER_S)],
                        out_hbm.at[cidx, pl.ds(sidx * _ROWS_PER_S, _ROWS_PER_S)])

    return k(x, src, dst, zeros)


_G = 10
_BLK = N // _G


def _mlp_stats_body(eps_ref, x_ref, parts_ref, w1_ref, b1_ref, w2_ref, b2_ref,
                    h_ref, s_ref, q_ref):
    i = pl.program_id(0)
    agg = parts_ref[0] + parts_ref[1]
    h0 = x_ref[...] * (1.0 + eps_ref[0]) + agg
    h1 = jnp.maximum(
        jnp.dot(h0, w1_ref[...], preferred_element_type=jnp.float32) + b1_ref[...],
        0.0)
    h2 = jnp.dot(h1.astype(jnp.bfloat16), w2_ref[...],
                 preferred_element_type=jnp.float32) + b2_ref[...]
    h_ref[...] = h2

    @pl.when(i == 0)
    def _():
        s_ref[...] = jnp.zeros_like(s_ref)
        q_ref[...] = jnp.zeros_like(q_ref)

    s_ref[...] += jnp.sum(h2, axis=0, keepdims=True)
    q_ref[...] += jnp.sum(h2 * h2, axis=0, keepdims=True)


def _bn_lin_body(h_ref, s_ref, q_ref, gamma_ref, beta_ref, lw_ref, lb_ref,
                 o_ref):
    mean = s_ref[...] * (1.0 / N)
    var = q_ref[...] * (1.0 / N) - mean * mean
    inv = lax.rsqrt(var + 1e-5)
    scale = gamma_ref[...] * inv
    shift = beta_ref[...] - mean * scale
    h3 = jnp.maximum(h_ref[...] * scale + shift, 0.0)
    o_ref[...] = (jnp.dot(h3.astype(jnp.bfloat16), lw_ref[...],
                          preferred_element_type=jnp.float32)
                  + lb_ref[...])


def kernel(x, edge_index, eps, W1, b1, W2, b2, gamma, beta, lin_W, lin_b):
    src = edge_index[0].reshape(_NUM_CHUNKS, _CHUNK)
    dst = edge_index[1].reshape(_NUM_CHUNKS, _CHUNK)
    zeros = jnp.zeros((_N_PAD, F), jnp.float32)
    parts = _sc_agg(x, src, dst, zeros)[:, :N, :]

    eps1 = jnp.reshape(eps, (1,))
    h2, s, q = pl.pallas_call(
        _mlp_stats_body,
        grid=(_G,),
        in_specs=[
            pl.BlockSpec(memory_space=pltpu.SMEM),
            pl.BlockSpec((_BLK, F), lambda i: (i, 0)),
            pl.BlockSpec((_NC, _BLK, F), lambda i: (0, i, 0)),
            pl.BlockSpec((F, H), lambda i: (0, 0)),
            pl.BlockSpec((1, H), lambda i: (0, 0)),
            pl.BlockSpec((H, H), lambda i: (0, 0)),
            pl.BlockSpec((1, H), lambda i: (0, 0)),
        ],
        out_specs=[
            pl.BlockSpec((_BLK, H), lambda i: (i, 0)),
            pl.BlockSpec((1, H), lambda i: (0, 0)),
            pl.BlockSpec((1, H), lambda i: (0, 0)),
        ],
        out_shape=[
            jax.ShapeDtypeStruct((N, H), jnp.float32),
            jax.ShapeDtypeStruct((1, H), jnp.float32),
            jax.ShapeDtypeStruct((1, H), jnp.float32),
        ],
    )(eps1, x, parts, W1, b1.reshape(1, H), W2.astype(jnp.bfloat16),
      b2.reshape(1, H))

    out = pl.pallas_call(
        _bn_lin_body,
        grid=(_G,),
        in_specs=[
            pl.BlockSpec((_BLK, H), lambda i: (i, 0)),
            pl.BlockSpec((1, H), lambda i: (0, 0)),
            pl.BlockSpec((1, H), lambda i: (0, 0)),
            pl.BlockSpec((1, H), lambda i: (0, 0)),
            pl.BlockSpec((1, H), lambda i: (0, 0)),
            pl.BlockSpec((H, H), lambda i: (0, 0)),
            pl.BlockSpec((1, H), lambda i: (0, 0)),
        ],
        out_specs=pl.BlockSpec((_BLK, H), lambda i: (i, 0)),
        out_shape=jax.ShapeDtypeStruct((N, H), jnp.float32),
    )(h2, s, q, gamma.reshape(1, H), beta.reshape(1, H),
      lin_W.astype(jnp.bfloat16), lin_b.reshape(1, H))
    return out


# speedup vs baseline: 7.3741x; 1.0751x over previous
import functools

import jax
import jax.numpy as jnp
from jax import lax
from jax.experimental import pallas as pl
from jax.experimental.pallas import tpu as pltpu
from jax.experimental.pallas import tpu_sc as plsc

N = 10000
F = 128
H = 512
E = 160000


_CHUNK = 125
_NUM_CHUNKS = E // _CHUNK
_NC = 2
_NS = 16
_NW = _NC * _NS
_CHUNKS_PER_W = _NUM_CHUNKS // _NW
_ROWS_PER_S = 632
_N_PAD = _NS * _ROWS_PER_S


_NBUF = 2
_ZROWS = 40


def _sc_agg(x, ei):
    mesh = plsc.VectorSubcoreMesh(core_axis_name="c", subcore_axis_name="s")

    @functools.partial(
        pl.kernel,
        mesh=mesh,
        out_type=jax.ShapeDtypeStruct((_NC, _N_PAD, F), jnp.float32),
        scratch_types=[
            pltpu.VMEM((_CHUNKS_PER_W, _CHUNK), jnp.int32),
            pltpu.VMEM((_CHUNKS_PER_W, _CHUNK), jnp.int32),
            pltpu.VMEM((_ZROWS, F), jnp.float32),
            pltpu.VMEM_SHARED((_N_PAD, F), jnp.float32),
        ]
        + [pltpu.VMEM((_CHUNK, F), jnp.float32) for _ in range(_NBUF)]
        + [pltpu.SemaphoreType.DMA for _ in range(2 * _NBUF)],
    )
    def k(x_hbm, ei_hbm, out_hbm, src_v, dst_v, zbuf, agg_sh, *bufs_sems):
        rows = bufs_sems[:_NBUF]
        gsem = bufs_sems[_NBUF:2 * _NBUF]
        ssem = bufs_sems[2 * _NBUF:]
        cidx = lax.axis_index("c")
        sidx = lax.axis_index("s")
        wid = sidx * _NC + cidx
        cbase = wid * _CHUNKS_PER_W

        pltpu.sync_copy(ei_hbm.at[0, pl.ds(cbase, _CHUNKS_PER_W)], src_v)
        pltpu.sync_copy(ei_hbm.at[1, pl.ds(cbase, _CHUNKS_PER_W)], dst_v)

        zv = jnp.zeros((16,), jnp.float32)

        def zbody(t, _):
            zbuf[t // 8, pl.ds((t % 8) * 16, 16)] = zv
            return ()

        lax.fori_loop(0, _ZROWS * (F // 16), zbody, (), unroll=False)
        sbase = sidx * _ROWS_PER_S
        for kk in range(_ROWS_PER_S // _ZROWS):
            pltpu.sync_copy(zbuf, agg_sh.at[pl.ds(sbase + kk * _ZROWS, _ZROWS)])
        _rem = _ROWS_PER_S % _ZROWS
        if _rem:
            pltpu.sync_copy(
                zbuf.at[pl.ds(0, _rem)],
                agg_sh.at[pl.ds(sbase + (_ROWS_PER_S // _ZROWS) * _ZROWS, _rem)])
        plsc.subcore_barrier()

        def gather(j, b):
            pltpu.async_copy(x_hbm.at[src_v.at[j]], rows[b], gsem[b])

        def gather_wait(j, b):
            pltpu.make_async_copy(x_hbm.at[src_v.at[j]], rows[b], gsem[b]).wait()

        def scatter(j, b):
            pltpu.async_copy(rows[b], agg_sh.at[dst_v.at[j]], ssem[b], add=True)

        def scatter_wait(j, b):
            pltpu.make_async_copy(rows[b], agg_sh.at[dst_v.at[j]],
                                  ssem[b]).wait()

        for b in range(_NBUF - 1):
            gather(b, b)

        def body(t, _):
            for b in range(_NBUF):
                j = _NBUF * t + b
                gather_wait(j, b)
                scatter(j, b)
                jn = j + _NBUF - 1
                bn = (b + _NBUF - 1) % _NBUF

                @pl.when(jnp.logical_and(jn < _CHUNKS_PER_W, jn >= _NBUF))
                def _():
                    scatter_wait(jn - _NBUF, bn)

                @pl.when(jn < _CHUNKS_PER_W)
                def _():
                    gather(jn, bn)
            return ()

        lax.fori_loop(0, _CHUNKS_PER_W // _NBUF, body, (), unroll=False)
        for j in range(_CHUNKS_PER_W - _NBUF, _CHUNKS_PER_W):
            scatter_wait(j, j % _NBUF)
        plsc.subcore_barrier()

        pltpu.sync_copy(agg_sh.at[pl.ds(sidx * _ROWS_PER_S, _ROWS_PER_S)],
                        out_hbm.at[cidx, pl.ds(sidx * _ROWS_PER_S, _ROWS_PER_S)])

    return k(x, ei)


_G = 10
_BLK = N // _G


def _mlp_stats_body(eps_ref, x_ref, parts_ref, w1_ref, b1_ref, w2_ref, b2_ref,
                    h_ref, s_ref, q_ref):
    i = pl.program_id(0)
    agg = parts_ref[0] + parts_ref[1]
    h0 = x_ref[...] * (1.0 + eps_ref[0]) + agg
    h1 = jnp.maximum(
        jnp.dot(h0, w1_ref[...], preferred_element_type=jnp.float32) + b1_ref[...],
        0.0)
    h2 = jnp.dot(h1.astype(jnp.bfloat16), w2_ref[...],
                 preferred_element_type=jnp.float32) + b2_ref[...]
    h_ref[...] = h2

    @pl.when(i == 0)
    def _():
        s_ref[...] = jnp.zeros_like(s_ref)
        q_ref[...] = jnp.zeros_like(q_ref)

    s_ref[...] += jnp.sum(h2, axis=0, keepdims=True)
    q_ref[...] += jnp.sum(h2 * h2, axis=0, keepdims=True)


def _bn_lin_body(h_ref, s_ref, q_ref, gamma_ref, beta_ref, lw_ref, lb_ref,
                 o_ref):
    mean = s_ref[...] * (1.0 / N)
    var = q_ref[...] * (1.0 / N) - mean * mean
    inv = lax.rsqrt(var + 1e-5)
    scale = gamma_ref[...] * inv
    shift = beta_ref[...] - mean * scale
    h3 = jnp.maximum(h_ref[...] * scale + shift, 0.0)
    o_ref[...] = (jnp.dot(h3.astype(jnp.bfloat16), lw_ref[...],
                          preferred_element_type=jnp.float32)
                  + lb_ref[...])


def kernel(x, edge_index, eps, W1, b1, W2, b2, gamma, beta, lin_W, lin_b):
    ei = edge_index.reshape(2, _NUM_CHUNKS, _CHUNK)
    parts = _sc_agg(x, ei)[:, :N, :]

    eps1 = jnp.reshape(eps, (1,))
    h2, s, q = pl.pallas_call(
        _mlp_stats_body,
        grid=(_G,),
        in_specs=[
            pl.BlockSpec(memory_space=pltpu.SMEM),
            pl.BlockSpec((_BLK, F), lambda i: (i, 0)),
            pl.BlockSpec((_NC, _BLK, F), lambda i: (0, i, 0)),
            pl.BlockSpec((F, H), lambda i: (0, 0)),
            pl.BlockSpec((1, H), lambda i: (0, 0)),
            pl.BlockSpec((H, H), lambda i: (0, 0)),
            pl.BlockSpec((1, H), lambda i: (0, 0)),
        ],
        out_specs=[
            pl.BlockSpec((_BLK, H), lambda i: (i, 0)),
            pl.BlockSpec((1, H), lambda i: (0, 0)),
            pl.BlockSpec((1, H), lambda i: (0, 0)),
        ],
        out_shape=[
            jax.ShapeDtypeStruct((N, H), jnp.float32),
            jax.ShapeDtypeStruct((1, H), jnp.float32),
            jax.ShapeDtypeStruct((1, H), jnp.float32),
        ],
    )(eps1, x, parts, W1, b1.reshape(1, H), W2.astype(jnp.bfloat16),
      b2.reshape(1, H))

    out = pl.pallas_call(
        _bn_lin_body,
        grid=(_G,),
        in_specs=[
            pl.BlockSpec((_BLK, H), lambda i: (i, 0)),
            pl.BlockSpec((1, H), lambda i: (0, 0)),
            pl.BlockSpec((1, H), lambda i: (0, 0)),
            pl.BlockSpec((1, H), lambda i: (0, 0)),
            pl.BlockSpec((1, H), lambda i: (0, 0)),
            pl.BlockSpec((H, H), lambda i: (0, 0)),
            pl.BlockSpec((1, H), lambda i: (0, 0)),
        ],
        out_specs=pl.BlockSpec((_BLK, H), lambda i: (i, 0)),
        out_shape=jax.ShapeDtypeStruct((N, H), jnp.float32),
    )(h2, s, q, gamma.reshape(1, H), beta.reshape(1, H),
      lin_W.astype(jnp.bfloat16), lin_b.reshape(1, H))
    return out


# speedup vs baseline: 8.4904x; 1.1514x over previous
import functools

import jax
import jax.numpy as jnp
from jax import lax
from jax.experimental import pallas as pl
from jax.experimental.pallas import tpu as pltpu
from jax.experimental.pallas import tpu_sc as plsc

N = 10000
F = 128
H = 512
E = 160000


_CHUNK = 125
_NUM_CHUNKS = E // _CHUNK
_NC = 2
_NS = 16
_NW = _NC * _NS
_CHUNKS_PER_W = _NUM_CHUNKS // _NW
_ROWS_PER_S = 632
_N_PAD = _NS * _ROWS_PER_S


_NBUF = 2
_ZROWS = 40


def _sc_agg(x, ei):
    mesh = plsc.VectorSubcoreMesh(core_axis_name="c", subcore_axis_name="s")

    @functools.partial(
        pl.kernel,
        mesh=mesh,
        out_type=jax.ShapeDtypeStruct((_NC, _N_PAD, F), jnp.float32),
        scratch_types=[
            pltpu.VMEM((_CHUNKS_PER_W, _CHUNK), jnp.int32),
            pltpu.VMEM((_CHUNKS_PER_W, _CHUNK), jnp.int32),
            pltpu.VMEM((_ZROWS, F), jnp.float32),
            pltpu.VMEM_SHARED((_N_PAD, F), jnp.float32),
        ]
        + [pltpu.VMEM((_CHUNK, F), jnp.float32) for _ in range(_NBUF)]
        + [pltpu.SemaphoreType.DMA for _ in range(2 * _NBUF)],
    )
    def k(x_hbm, ei_hbm, out_hbm, src_v, dst_v, zbuf, agg_sh, *bufs_sems):
        rows = bufs_sems[:_NBUF]
        gsem = bufs_sems[_NBUF:2 * _NBUF]
        ssem = bufs_sems[2 * _NBUF:]
        cidx = lax.axis_index("c")
        sidx = lax.axis_index("s")
        wid = sidx * _NC + cidx
        cbase = wid * _CHUNKS_PER_W

        pltpu.sync_copy(ei_hbm.at[0, pl.ds(cbase, _CHUNKS_PER_W)], src_v)
        pltpu.sync_copy(ei_hbm.at[1, pl.ds(cbase, _CHUNKS_PER_W)], dst_v)

        zv = jnp.zeros((16,), jnp.float32)

        def zbody(t, _):
            zbuf[t // 8, pl.ds((t % 8) * 16, 16)] = zv
            return ()

        lax.fori_loop(0, _ZROWS * (F // 16), zbody, (), unroll=False)
        sbase = sidx * _ROWS_PER_S
        for kk in range(_ROWS_PER_S // _ZROWS):
            pltpu.sync_copy(zbuf, agg_sh.at[pl.ds(sbase + kk * _ZROWS, _ZROWS)])
        _rem = _ROWS_PER_S % _ZROWS
        if _rem:
            pltpu.sync_copy(
                zbuf.at[pl.ds(0, _rem)],
                agg_sh.at[pl.ds(sbase + (_ROWS_PER_S // _ZROWS) * _ZROWS, _rem)])
        plsc.subcore_barrier()

        def gather(j, b):
            pltpu.async_copy(x_hbm.at[src_v.at[j]], rows[b], gsem[b])

        def gather_wait(j, b):
            pltpu.make_async_copy(x_hbm.at[src_v.at[j]], rows[b], gsem[b]).wait()

        def scatter(j, b):
            pltpu.async_copy(rows[b], agg_sh.at[dst_v.at[j]], ssem[b], add=True)

        def scatter_wait(j, b):
            pltpu.make_async_copy(rows[b], agg_sh.at[dst_v.at[j]],
                                  ssem[b]).wait()

        for b in range(_NBUF - 1):
            gather(b, b)

        def body(t, _):
            for b in range(_NBUF):
                j = _NBUF * t + b
                gather_wait(j, b)
                scatter(j, b)
                jn = j + _NBUF - 1
                bn = (b + _NBUF - 1) % _NBUF

                @pl.when(jnp.logical_and(jn < _CHUNKS_PER_W, jn >= _NBUF))
                def _():
                    scatter_wait(jn - _NBUF, bn)

                @pl.when(jn < _CHUNKS_PER_W)
                def _():
                    gather(jn, bn)
            return ()

        lax.fori_loop(0, _CHUNKS_PER_W // _NBUF, body, (), unroll=False)
        for j in range(_CHUNKS_PER_W - _NBUF, _CHUNKS_PER_W):
            scatter_wait(j, j % _NBUF)
        plsc.subcore_barrier()

        pltpu.sync_copy(agg_sh.at[pl.ds(sidx * _ROWS_PER_S, _ROWS_PER_S)],
                        out_hbm.at[cidx, pl.ds(sidx * _ROWS_PER_S, _ROWS_PER_S)])

    return k(x, ei)


_G = 10
_BLK = N // _G


def _fused_tc_body(eps_ref, x_ref, parts_ref, w1_ref, b1_ref, w2_ref, b2_ref,
                   gamma_ref, beta_ref, lw_ref, lb_ref, o_ref,
                   h2_scr, s_scr, q_scr):
    p = pl.program_id(0)
    i = pl.program_id(1)

    @pl.when(p == 0)
    def _():
        agg = parts_ref[0] + parts_ref[1]
        h0 = x_ref[...] * (1.0 + eps_ref[0]) + agg
        h1 = jnp.maximum(
            jnp.dot(h0, w1_ref[...], preferred_element_type=jnp.float32)
            + b1_ref[...], 0.0)
        h2 = jnp.dot(h1.astype(jnp.bfloat16),
                     w2_ref[...].astype(jnp.bfloat16),
                     preferred_element_type=jnp.float32) + b2_ref[...]
        h2_scr[pl.ds(i * _BLK, _BLK), :] = h2

        @pl.when(i == 0)
        def _():
            s_scr[...] = jnp.zeros_like(s_scr)
            q_scr[...] = jnp.zeros_like(q_scr)

        s_scr[...] += jnp.sum(h2, axis=0, keepdims=True)
        q_scr[...] += jnp.sum(h2 * h2, axis=0, keepdims=True)

    @pl.when(p == 1)
    def _():
        mean = s_scr[...] * (1.0 / N)
        var = q_scr[...] * (1.0 / N) - mean * mean
        inv = lax.rsqrt(var + 1e-5)
        scale = gamma_ref[...] * inv
        shift = beta_ref[...] - mean * scale
        h2 = h2_scr[pl.ds(i * _BLK, _BLK), :]
        h3 = jnp.maximum(h2 * scale + shift, 0.0)
        o_ref[...] = (jnp.dot(h3.astype(jnp.bfloat16),
                              lw_ref[...].astype(jnp.bfloat16),
                              preferred_element_type=jnp.float32)
                      + lb_ref[...])


def kernel(x, edge_index, eps, W1, b1, W2, b2, gamma, beta, lin_W, lin_b):
    ei = edge_index.reshape(2, _NUM_CHUNKS, _CHUNK)
    parts = _sc_agg(x, ei)

    eps1 = jnp.reshape(eps, (1,))
    const = lambda p, i: (0, 0)
    out = pl.pallas_call(
        _fused_tc_body,
        grid=(2, _G),
        in_specs=[
            pl.BlockSpec(memory_space=pltpu.SMEM),
            pl.BlockSpec((_BLK, F), lambda p, i: (i * (1 - p), 0)),
            pl.BlockSpec((_NC, _BLK, F), lambda p, i: (0, i * (1 - p), 0)),
            pl.BlockSpec((F, H), const),
            pl.BlockSpec((1, H), const),
            pl.BlockSpec((H, H), const),
            pl.BlockSpec((1, H), const),
            pl.BlockSpec((1, H), const),
            pl.BlockSpec((1, H), const),
            pl.BlockSpec((H, H), const),
            pl.BlockSpec((1, H), const),
        ],
        out_specs=pl.BlockSpec((_BLK, H), lambda p, i: (i * p, 0)),
        out_shape=jax.ShapeDtypeStruct((N, H), jnp.float32),
        scratch_shapes=[
            pltpu.VMEM((N, H), jnp.float32),
            pltpu.VMEM((1, H), jnp.float32),
            pltpu.VMEM((1, H), jnp.float32),
        ],
    )(eps1, x, parts, W1, b1.reshape(1, H), W2, b2.reshape(1, H),
      gamma.reshape(1, H), beta.reshape(1, H), lin_W, lin_b.reshape(1, H))
    return out


# speedup vs baseline: 9.3363x; 1.0996x over previous
import functools

import jax
import jax.numpy as jnp
from jax import lax
from jax.experimental import pallas as pl
from jax.experimental.pallas import tpu as pltpu
from jax.experimental.pallas import tpu_sc as plsc

N = 10000
F = 128
H = 512
E = 160000


_CHUNK = 125
_NUM_CHUNKS = E // _CHUNK
_NC = 2
_NS = 16
_NW = _NC * _NS
_CHUNKS_PER_W = _NUM_CHUNKS // _NW
_ROWS_PER_S = 632
_N_PAD = _NS * _ROWS_PER_S


_NBUF = 2
_ZROWS = 40


def _sc_agg(x, ei):
    mesh = plsc.VectorSubcoreMesh(core_axis_name="c", subcore_axis_name="s")

    @functools.partial(
        pl.kernel,
        mesh=mesh,
        out_type=jax.ShapeDtypeStruct((_NC, _N_PAD, F), jnp.float32),
        scratch_types=[
            pltpu.VMEM((_CHUNKS_PER_W, _CHUNK), jnp.int32),
            pltpu.VMEM((_CHUNKS_PER_W, _CHUNK), jnp.int32),
            pltpu.VMEM((_ZROWS, F), jnp.float32),
            pltpu.VMEM_SHARED((_N_PAD, F), jnp.float32),
        ]
        + [pltpu.VMEM((_CHUNK, F), jnp.float32) for _ in range(_NBUF)]
        + [pltpu.SemaphoreType.DMA for _ in range(2 * _NBUF)],
    )
    def k(x_hbm, ei_hbm, out_hbm, src_v, dst_v, zbuf, agg_sh, *bufs_sems):
        rows = bufs_sems[:_NBUF]
        gsem = bufs_sems[_NBUF:2 * _NBUF]
        ssem = bufs_sems[2 * _NBUF:]
        cidx = lax.axis_index("c")
        sidx = lax.axis_index("s")
        wid = sidx * _NC + cidx
        cbase = wid * _CHUNKS_PER_W

        pltpu.sync_copy(ei_hbm.at[0, pl.ds(cbase, _CHUNKS_PER_W)], src_v)
        pltpu.sync_copy(ei_hbm.at[1, pl.ds(cbase, _CHUNKS_PER_W)], dst_v)

        zv = jnp.zeros((16,), jnp.float32)

        def zbody(t, _):
            zbuf[t // 8, pl.ds((t % 8) * 16, 16)] = zv
            return ()

        lax.fori_loop(0, _ZROWS * (F // 16), zbody, (), unroll=False)
        sbase = sidx * _ROWS_PER_S
        for kk in range(_ROWS_PER_S // _ZROWS):
            pltpu.sync_copy(zbuf, agg_sh.at[pl.ds(sbase + kk * _ZROWS, _ZROWS)])
        _rem = _ROWS_PER_S % _ZROWS
        if _rem:
            pltpu.sync_copy(
                zbuf.at[pl.ds(0, _rem)],
                agg_sh.at[pl.ds(sbase + (_ROWS_PER_S // _ZROWS) * _ZROWS, _rem)])
        plsc.subcore_barrier()

        def gather(j, b):
            pltpu.async_copy(x_hbm.at[src_v.at[j]], rows[b], gsem[b])

        def gather_wait(j, b):
            pltpu.make_async_copy(x_hbm.at[src_v.at[j]], rows[b], gsem[b]).wait()

        def scatter(j, b):
            pltpu.async_copy(rows[b], agg_sh.at[dst_v.at[j]], ssem[b], add=True)

        def scatter_wait(j, b):
            pltpu.make_async_copy(rows[b], agg_sh.at[dst_v.at[j]],
                                  ssem[b]).wait()

        for b in range(_NBUF):
            gather(b, b)

        def body(t, _):
            for b in range(_NBUF):
                j = _NBUF * t + b
                gather_wait(j, b)
                scatter(j, b)
                scatter_wait(j, b)

                @pl.when(j + _NBUF < _CHUNKS_PER_W)
                def _():
                    gather(j + _NBUF, b)
            return ()

        lax.fori_loop(0, _CHUNKS_PER_W // _NBUF, body, (), unroll=False)
        plsc.subcore_barrier()

        pltpu.sync_copy(agg_sh.at[pl.ds(sidx * _ROWS_PER_S, _ROWS_PER_S)],
                        out_hbm.at[cidx, pl.ds(sidx * _ROWS_PER_S, _ROWS_PER_S)])

    return k(x, ei)


_G = 10
_BLK = N // _G


def _fused_tc_body(eps_ref, x_ref, parts_ref, w1_ref, b1_ref, w2_ref, b2_ref,
                   gamma_ref, beta_ref, lw_ref, lb_ref, o_ref,
                   h2_scr, s_scr, q_scr):
    p = pl.program_id(0)
    i = pl.program_id(1)

    @pl.when(p == 0)
    def _():
        agg = parts_ref[0] + parts_ref[1]
        h0 = x_ref[...] * (1.0 + eps_ref[0]) + agg
        h1 = jnp.maximum(
            jnp.dot(h0, w1_ref[...], preferred_element_type=jnp.float32)
            + b1_ref[...], 0.0)
        h2 = jnp.dot(h1.astype(jnp.bfloat16),
                     w2_ref[...].astype(jnp.bfloat16),
                     preferred_element_type=jnp.float32) + b2_ref[...]
        h2_scr[pl.ds(i * _BLK, _BLK), :] = h2

        @pl.when(i == 0)
        def _():
            s_scr[...] = jnp.zeros_like(s_scr)
            q_scr[...] = jnp.zeros_like(q_scr)

        s_scr[...] += jnp.sum(h2, axis=0, keepdims=True)
        q_scr[...] += jnp.sum(h2 * h2, axis=0, keepdims=True)

    @pl.when(p == 1)
    def _():
        mean = s_scr[...] * (1.0 / N)
        var = q_scr[...] * (1.0 / N) - mean * mean
        inv = lax.rsqrt(var + 1e-5)
        scale = gamma_ref[...] * inv
        shift = beta_ref[...] - mean * scale
        h2 = h2_scr[pl.ds(i * _BLK, _BLK), :]
        h3 = jnp.maximum(h2 * scale + shift, 0.0)
        o_ref[...] = (jnp.dot(h3.astype(jnp.bfloat16),
                              lw_ref[...].astype(jnp.bfloat16),
                              preferred_element_type=jnp.float32)
                      + lb_ref[...])


def kernel(x, edge_index, eps, W1, b1, W2, b2, gamma, beta, lin_W, lin_b):
    ei = edge_index.reshape(2, _NUM_CHUNKS, _CHUNK)
    parts = _sc_agg(x, ei)

    eps1 = jnp.reshape(eps, (1,))
    const = lambda p, i: (0, 0)
    out = pl.pallas_call(
        _fused_tc_body,
        grid=(2, _G),
        in_specs=[
            pl.BlockSpec(memory_space=pltpu.SMEM),
            pl.BlockSpec((_BLK, F), lambda p, i: (i * (1 - p), 0)),
            pl.BlockSpec((_NC, _BLK, F), lambda p, i: (0, i * (1 - p), 0)),
            pl.BlockSpec((F, H), const),
            pl.BlockSpec((1, H), const),
            pl.BlockSpec((H, H), const),
            pl.BlockSpec((1, H), const),
            pl.BlockSpec((1, H), const),
            pl.BlockSpec((1, H), const),
            pl.BlockSpec((H, H), const),
            pl.BlockSpec((1, H), const),
        ],
        out_specs=pl.BlockSpec((_BLK, H), lambda p, i: (i * p, 0)),
        out_shape=jax.ShapeDtypeStruct((N, H), jnp.float32),
        scratch_shapes=[
            pltpu.VMEM((N, H), jnp.float32),
            pltpu.VMEM((1, H), jnp.float32),
            pltpu.VMEM((1, H), jnp.float32),
        ],
    )(eps1, x, parts, W1, b1.reshape(1, H), W2, b2.reshape(1, H),
      gamma.reshape(1, H), beta.reshape(1, H), lin_W, lin_b.reshape(1, H))
    return out


# speedup vs baseline: 9.5495x; 1.0228x over previous
import functools

import jax
import jax.numpy as jnp
from jax import lax
from jax.experimental import pallas as pl
from jax.experimental.pallas import tpu as pltpu
from jax.experimental.pallas import tpu_sc as plsc

N = 10000
F = 128
H = 512
E = 160000


_CHUNK = 125
_NUM_CHUNKS = E // _CHUNK
_NC = 2
_NS = 16
_NW = _NC * _NS
_CHUNKS_PER_W = _NUM_CHUNKS // _NW
_ROWS_PER_S = 632
_N_PAD = _NS * _ROWS_PER_S


_NBUF = 2
_ZROWS = 40


def _sc_agg(x, ei):
    mesh = plsc.VectorSubcoreMesh(core_axis_name="c", subcore_axis_name="s")

    @functools.partial(
        pl.kernel,
        mesh=mesh,
        out_type=jax.ShapeDtypeStruct((_NC, _N_PAD, F), jnp.float32),
        scratch_types=[
            pltpu.VMEM((_CHUNKS_PER_W, _CHUNK), jnp.int32),
            pltpu.VMEM((_CHUNKS_PER_W, _CHUNK), jnp.int32),
            pltpu.VMEM((_ZROWS, F), jnp.float32),
            pltpu.VMEM_SHARED((_N_PAD, F), jnp.float32),
        ]
        + [pltpu.VMEM((_CHUNK, F), jnp.float32) for _ in range(_NBUF)]
        + [pltpu.SemaphoreType.DMA for _ in range(2 * _NBUF)],
    )
    def k(x_hbm, ei_hbm, out_hbm, src_v, dst_v, zbuf, agg_sh, *bufs_sems):
        rows = bufs_sems[:_NBUF]
        gsem = bufs_sems[_NBUF:2 * _NBUF]
        ssem = bufs_sems[2 * _NBUF:]
        cidx = lax.axis_index("c")
        sidx = lax.axis_index("s")
        wid = sidx * _NC + cidx
        cbase = wid * _CHUNKS_PER_W

        pltpu.sync_copy(ei_hbm.at[0, pl.ds(cbase, _CHUNKS_PER_W)], src_v)
        pltpu.sync_copy(ei_hbm.at[1, pl.ds(cbase, _CHUNKS_PER_W)], dst_v)

        zv = jnp.zeros((16,), jnp.float32)

        def zbody(t, _):
            zbuf[t // 8, pl.ds((t % 8) * 16, 16)] = zv
            return ()

        lax.fori_loop(0, _ZROWS * (F // 16), zbody, (), unroll=False)
        sbase = sidx * _ROWS_PER_S
        for kk in range(_ROWS_PER_S // _ZROWS):
            pltpu.sync_copy(zbuf, agg_sh.at[pl.ds(sbase + kk * _ZROWS, _ZROWS)])
        _rem = _ROWS_PER_S % _ZROWS
        if _rem:
            pltpu.sync_copy(
                zbuf.at[pl.ds(0, _rem)],
                agg_sh.at[pl.ds(sbase + (_ROWS_PER_S // _ZROWS) * _ZROWS, _rem)])
        plsc.subcore_barrier()

        def gather(j, b):
            pltpu.async_copy(x_hbm.at[src_v.at[j]], rows[b], gsem[b])

        def gather_wait(j, b):
            pltpu.make_async_copy(x_hbm.at[src_v.at[j]], rows[b], gsem[b]).wait()

        def scatter(j, b):
            pltpu.async_copy(rows[b], agg_sh.at[dst_v.at[j]], ssem[b], add=True)

        def scatter_wait(j, b):
            pltpu.make_async_copy(rows[b], agg_sh.at[dst_v.at[j]],
                                  ssem[b]).wait()

        for b in range(_NBUF):
            gather(b, b)

        def body(t, _):
            for b in range(_NBUF):
                j = _NBUF * t + b
                gather_wait(j, b)
                scatter(j, b)
                scatter_wait(j, b)

                @pl.when(j + _NBUF < _CHUNKS_PER_W)
                def _():
                    gather(j + _NBUF, b)
            return ()

        lax.fori_loop(0, _CHUNKS_PER_W // _NBUF, body, (), unroll=False)
        plsc.subcore_barrier()

        pltpu.sync_copy(agg_sh.at[pl.ds(sidx * _ROWS_PER_S, _ROWS_PER_S)],
                        out_hbm.at[cidx, pl.ds(sidx * _ROWS_PER_S, _ROWS_PER_S)])

    return k(x, ei)


_G = 5
_BLK = N // _G


def _fused_tc_body(eps_ref, x_ref, parts_ref, w1_ref, b1_ref, w2_ref, b2_ref,
                   gamma_ref, beta_ref, lw_ref, lb_ref, o_ref,
                   h2_scr, s_scr, q_scr):
    p = pl.program_id(0)
    i = pl.program_id(1)

    @pl.when(p == 0)
    def _():
        agg = parts_ref[0] + parts_ref[1]
        h0 = x_ref[...] * (1.0 + eps_ref[0]) + agg
        h1 = jnp.maximum(
            jnp.dot(h0, w1_ref[...], preferred_element_type=jnp.float32)
            + b1_ref[...], 0.0)
        h2 = jnp.dot(h1.astype(jnp.bfloat16),
                     w2_ref[...].astype(jnp.bfloat16),
                     preferred_element_type=jnp.float32) + b2_ref[...]
        h2_scr[pl.ds(i * _BLK, _BLK), :] = h2

        @pl.when(i == 0)
        def _():
            s_scr[...] = jnp.zeros_like(s_scr)
            q_scr[...] = jnp.zeros_like(q_scr)

        s_scr[...] += jnp.sum(h2, axis=0, keepdims=True)
        q_scr[...] += jnp.sum(h2 * h2, axis=0, keepdims=True)

    @pl.when(p == 1)
    def _():
        mean = s_scr[...] * (1.0 / N)
        var = q_scr[...] * (1.0 / N) - mean * mean
        inv = lax.rsqrt(var + 1e-5)
        scale = gamma_ref[...] * inv
        shift = beta_ref[...] - mean * scale
        h2 = h2_scr[pl.ds(i * _BLK, _BLK), :]
        h3 = jnp.maximum(h2 * scale + shift, 0.0)
        o_ref[...] = (jnp.dot(h3.astype(jnp.bfloat16),
                              lw_ref[...].astype(jnp.bfloat16),
                              preferred_element_type=jnp.float32)
                      + lb_ref[...])


def kernel(x, edge_index, eps, W1, b1, W2, b2, gamma, beta, lin_W, lin_b):
    ei = edge_index.reshape(2, _NUM_CHUNKS, _CHUNK)
    parts = _sc_agg(x, ei)

    eps1 = jnp.reshape(eps, (1,))
    const = lambda p, i: (0, 0)
    out = pl.pallas_call(
        _fused_tc_body,
        grid=(2, _G),
        in_specs=[
            pl.BlockSpec(memory_space=pltpu.SMEM),
            pl.BlockSpec((_BLK, F), lambda p, i: (i * (1 - p), 0)),
            pl.BlockSpec((_NC, _BLK, F), lambda p, i: (0, i * (1 - p), 0)),
            pl.BlockSpec((F, H), const),
            pl.BlockSpec((1, H), const),
            pl.BlockSpec((H, H), const),
            pl.BlockSpec((1, H), const),
            pl.BlockSpec((1, H), const),
            pl.BlockSpec((1, H), const),
            pl.BlockSpec((H, H), const),
            pl.BlockSpec((1, H), const),
        ],
        out_specs=pl.BlockSpec((_BLK, H), lambda p, i: (i * p, 0)),
        out_shape=jax.ShapeDtypeStruct((N, H), jnp.float32),
        scratch_shapes=[
            pltpu.VMEM((N, H), jnp.float32),
            pltpu.VMEM((1, H), jnp.float32),
            pltpu.VMEM((1, H), jnp.float32),
        ],
    )(eps1, x, parts, W1, b1.reshape(1, H), W2, b2.reshape(1, H),
      gamma.reshape(1, H), beta.reshape(1, H), lin_W, lin_b.reshape(1, H))
    return out


# speedup vs baseline: 9.5514x; 1.0002x over previous
import functools

import jax
import jax.numpy as jnp
from jax import lax
from jax.experimental import pallas as pl
from jax.experimental.pallas import tpu as pltpu
from jax.experimental.pallas import tpu_sc as plsc

N = 10000
F = 128
H = 512
E = 160000


_CHUNK = 125
_NUM_CHUNKS = E // _CHUNK
_NC = 2
_NS = 16
_NW = _NC * _NS
_CHUNKS_PER_W = _NUM_CHUNKS // _NW
_ROWS_PER_S = 632
_N_PAD = _NS * _ROWS_PER_S


_NBUF = 2
_ZROWS = 40


def _sc_agg(x, ei):
    mesh = plsc.VectorSubcoreMesh(core_axis_name="c", subcore_axis_name="s")

    @functools.partial(
        pl.kernel,
        mesh=mesh,
        out_type=jax.ShapeDtypeStruct((_NC, _N_PAD, F), jnp.float32),
        scratch_types=[
            pltpu.VMEM((_CHUNKS_PER_W, _CHUNK), jnp.int32),
            pltpu.VMEM((_CHUNKS_PER_W, _CHUNK), jnp.int32),
            pltpu.VMEM((_ZROWS, F), jnp.float32),
            pltpu.VMEM_SHARED((_N_PAD, F), jnp.float32),
        ]
        + [pltpu.VMEM((_CHUNK, F), jnp.float32) for _ in range(_NBUF)]
        + [pltpu.SemaphoreType.DMA for _ in range(2 * _NBUF)],
    )
    def k(x_hbm, ei_hbm, out_hbm, src_v, dst_v, zbuf, agg_sh, *bufs_sems):
        rows = bufs_sems[:_NBUF]
        gsem = bufs_sems[_NBUF:2 * _NBUF]
        ssem = bufs_sems[2 * _NBUF:]
        cidx = lax.axis_index("c")
        sidx = lax.axis_index("s")
        wid = sidx * _NC + cidx
        cbase = wid * _CHUNKS_PER_W

        pltpu.sync_copy(ei_hbm.at[0, pl.ds(cbase, _CHUNKS_PER_W)], src_v)
        pltpu.sync_copy(ei_hbm.at[1, pl.ds(cbase, _CHUNKS_PER_W)], dst_v)

        zv = jnp.zeros((16,), jnp.float32)

        def zbody(t, _):
            zbuf[t // 8, pl.ds((t % 8) * 16, 16)] = zv
            return ()

        lax.fori_loop(0, _ZROWS * (F // 16), zbody, (), unroll=False)
        sbase = sidx * _ROWS_PER_S
        for kk in range(_ROWS_PER_S // _ZROWS):
            pltpu.sync_copy(zbuf, agg_sh.at[pl.ds(sbase + kk * _ZROWS, _ZROWS)])
        _rem = _ROWS_PER_S % _ZROWS
        if _rem:
            pltpu.sync_copy(
                zbuf.at[pl.ds(0, _rem)],
                agg_sh.at[pl.ds(sbase + (_ROWS_PER_S // _ZROWS) * _ZROWS, _rem)])
        plsc.subcore_barrier()

        _H0 = 64
        _H1 = _CHUNK - _H0

        def gather(j, b):
            pltpu.async_copy(x_hbm.at[src_v.at[j, pl.ds(0, _H0)]],
                             rows[b].at[pl.ds(0, _H0)], gsem[b])
            pltpu.async_copy(x_hbm.at[src_v.at[j, pl.ds(_H0, _H1)]],
                             rows[b].at[pl.ds(_H0, _H1)], gsem[b])

        def gather_wait(j, b):
            pltpu.make_async_copy(x_hbm.at[src_v.at[j, pl.ds(0, _H0)]],
                                  rows[b].at[pl.ds(0, _H0)], gsem[b]).wait()
            pltpu.make_async_copy(x_hbm.at[src_v.at[j, pl.ds(_H0, _H1)]],
                                  rows[b].at[pl.ds(_H0, _H1)], gsem[b]).wait()

        def scatter(j, b):
            pltpu.async_copy(rows[b], agg_sh.at[dst_v.at[j]], ssem[b], add=True)

        def scatter_wait(j, b):
            pltpu.make_async_copy(rows[b], agg_sh.at[dst_v.at[j]],
                                  ssem[b]).wait()

        for b in range(_NBUF):
            gather(b, b)

        def body(t, _):
            for b in range(_NBUF):
                j = _NBUF * t + b
                gather_wait(j, b)
                scatter(j, b)
                scatter_wait(j, b)

                @pl.when(j + _NBUF < _CHUNKS_PER_W)
                def _():
                    gather(j + _NBUF, b)
            return ()

        lax.fori_loop(0, _CHUNKS_PER_W // _NBUF, body, (), unroll=False)
        plsc.subcore_barrier()

        pltpu.sync_copy(agg_sh.at[pl.ds(sidx * _ROWS_PER_S, _ROWS_PER_S)],
                        out_hbm.at[cidx, pl.ds(sidx * _ROWS_PER_S, _ROWS_PER_S)])

    return k(x, ei)


_G = 5
_BLK = N // _G


def _fused_tc_body(eps_ref, x_ref, parts_ref, w1_ref, b1_ref, w2_ref, b2_ref,
                   gamma_ref, beta_ref, lw_ref, lb_ref, o_ref,
                   h2_scr, s_scr, q_scr):
    p = pl.program_id(0)
    i = pl.program_id(1)

    @pl.when(p == 0)
    def _():
        agg = parts_ref[0] + parts_ref[1]
        h0 = x_ref[...] * (1.0 + eps_ref[0]) + agg
        h1 = jnp.maximum(
            jnp.dot(h0, w1_ref[...], preferred_element_type=jnp.float32)
            + b1_ref[...], 0.0)
        h2 = jnp.dot(h1.astype(jnp.bfloat16),
                     w2_ref[...].astype(jnp.bfloat16),
                     preferred_element_type=jnp.float32) + b2_ref[...]
        h2_scr[pl.ds(i * _BLK, _BLK), :] = h2

        @pl.when(i == 0)
        def _():
            s_scr[...] = jnp.zeros_like(s_scr)
            q_scr[...] = jnp.zeros_like(q_scr)

        s_scr[...] += jnp.sum(h2, axis=0, keepdims=True)
        q_scr[...] += jnp.sum(h2 * h2, axis=0, keepdims=True)

    @pl.when(p == 1)
    def _():
        mean = s_scr[...] * (1.0 / N)
        var = q_scr[...] * (1.0 / N) - mean * mean
        inv = lax.rsqrt(var + 1e-5)
        scale = gamma_ref[...] * inv
        shift = beta_ref[...] - mean * scale
        h2 = h2_scr[pl.ds(i * _BLK, _BLK), :]
        h3 = jnp.maximum(h2 * scale + shift, 0.0)
        o_ref[...] = (jnp.dot(h3.astype(jnp.bfloat16),
                              lw_ref[...].astype(jnp.bfloat16),
                              preferred_element_type=jnp.float32)
                      + lb_ref[...])


def kernel(x, edge_index, eps, W1, b1, W2, b2, gamma, beta, lin_W, lin_b):
    ei = edge_index.reshape(2, _NUM_CHUNKS, _CHUNK)
    parts = _sc_agg(x, ei)

    eps1 = jnp.reshape(eps, (1,))
    const = lambda p, i: (0, 0)
    out = pl.pallas_call(
        _fused_tc_body,
        grid=(2, _G),
        in_specs=[
            pl.BlockSpec(memory_space=pltpu.SMEM),
            pl.BlockSpec((_BLK, F), lambda p, i: (i * (1 - p), 0)),
            pl.BlockSpec((_NC, _BLK, F), lambda p, i: (0, i * (1 - p), 0)),
            pl.BlockSpec((F, H), const),
            pl.BlockSpec((1, H), const),
            pl.BlockSpec((H, H), const),
            pl.BlockSpec((1, H), const),
            pl.BlockSpec((1, H), const),
            pl.BlockSpec((1, H), const),
            pl.BlockSpec((H, H), const),
            pl.BlockSpec((1, H), const),
        ],
        out_specs=pl.BlockSpec((_BLK, H), lambda p, i: (i * p, 0)),
        out_shape=jax.ShapeDtypeStruct((N, H), jnp.float32),
        scratch_shapes=[
            pltpu.VMEM((N, H), jnp.float32),
            pltpu.VMEM((1, H), jnp.float32),
            pltpu.VMEM((1, H), jnp.float32),
        ],
    )(eps1, x, parts, W1, b1.reshape(1, H), W2, b2.reshape(1, H),
      gamma.reshape(1, H), beta.reshape(1, H), lin_W, lin_b.reshape(1, H))
    return out


# speedup vs baseline: 9.7404x; 1.0198x over previous
import functools

import jax
import jax.numpy as jnp
from jax import lax
from jax.experimental import pallas as pl
from jax.experimental.pallas import tpu as pltpu
from jax.experimental.pallas import tpu_sc as plsc

N = 10000
F = 128
H = 512
E = 160000


_CHUNK = 125
_NUM_CHUNKS = E // _CHUNK
_NC = 2
_NS = 16
_NW = _NC * _NS
_CHUNKS_PER_W = _NUM_CHUNKS // _NW
_ROWS_PER_S = 632
_N_PAD = _NS * _ROWS_PER_S


_NBUF = 2
_ZROWS = 40


def _sc_agg(x, ei):
    mesh = plsc.VectorSubcoreMesh(core_axis_name="c", subcore_axis_name="s")

    @functools.partial(
        pl.kernel,
        mesh=mesh,
        out_type=jax.ShapeDtypeStruct((_NC, _N_PAD, F), jnp.float32),
        scratch_types=[
            pltpu.VMEM((_CHUNKS_PER_W, _CHUNK), jnp.int32),
            pltpu.VMEM((_CHUNKS_PER_W, _CHUNK), jnp.int32),
            pltpu.VMEM((_ZROWS, F), jnp.float32),
            pltpu.VMEM_SHARED((_N_PAD, F), jnp.float32),
        ]
        + [pltpu.VMEM((_CHUNK, F), jnp.float32) for _ in range(_NBUF)]
        + [pltpu.SemaphoreType.DMA for _ in range(2 * _NBUF)],
    )
    def k(x_hbm, ei_hbm, out_hbm, src_v, dst_v, zbuf, agg_sh, *bufs_sems):
        rows = bufs_sems[:_NBUF]
        gsem = bufs_sems[_NBUF:2 * _NBUF]
        ssem = bufs_sems[2 * _NBUF:]
        cidx = lax.axis_index("c")
        sidx = lax.axis_index("s")
        wid = sidx * _NC + cidx
        cbase = wid * _CHUNKS_PER_W

        pltpu.sync_copy(ei_hbm.at[0, pl.ds(cbase, _CHUNKS_PER_W)], src_v)
        pltpu.sync_copy(ei_hbm.at[1, pl.ds(cbase, _CHUNKS_PER_W)], dst_v)

        for b in range(_NBUF):
            pltpu.async_copy(x_hbm.at[src_v.at[b]], rows[b], gsem[b])

        zv = jnp.zeros((16,), jnp.float32)

        def zbody(t, _):
            zbuf[t // 8, pl.ds((t % 8) * 16, 16)] = zv
            return ()

        lax.fori_loop(0, _ZROWS * (F // 16), zbody, (), unroll=False)
        sbase = sidx * _ROWS_PER_S
        for kk in range(_ROWS_PER_S // _ZROWS):
            pltpu.sync_copy(zbuf, agg_sh.at[pl.ds(sbase + kk * _ZROWS, _ZROWS)])
        _rem = _ROWS_PER_S % _ZROWS
        if _rem:
            pltpu.sync_copy(
                zbuf.at[pl.ds(0, _rem)],
                agg_sh.at[pl.ds(sbase + (_ROWS_PER_S // _ZROWS) * _ZROWS, _rem)])
        plsc.subcore_barrier()

        def gather(j, b):
            pltpu.async_copy(x_hbm.at[src_v.at[j]], rows[b], gsem[b])

        def gather_wait(j, b):
            pltpu.make_async_copy(x_hbm.at[src_v.at[j]], rows[b], gsem[b]).wait()

        def scatter(j, b):
            pltpu.async_copy(rows[b], agg_sh.at[dst_v.at[j]], ssem[b], add=True)

        def scatter_wait(j, b):
            pltpu.make_async_copy(rows[b], agg_sh.at[dst_v.at[j]],
                                  ssem[b]).wait()

        def body(t, _):
            for b in range(_NBUF):
                j = _NBUF * t + b
                gather_wait(j, b)
                scatter(j, b)
                scatter_wait(j, b)

                @pl.when(j + _NBUF < _CHUNKS_PER_W)
                def _():
                    gather(j + _NBUF, b)
            return ()

        lax.fori_loop(0, _CHUNKS_PER_W // _NBUF, body, (), unroll=False)
        plsc.subcore_barrier()

        pltpu.sync_copy(agg_sh.at[pl.ds(sidx * _ROWS_PER_S, _ROWS_PER_S)],
                        out_hbm.at[cidx, pl.ds(sidx * _ROWS_PER_S, _ROWS_PER_S)])

    return k(x, ei)


_G = 5
_BLK = N // _G


def _fused_tc_body(eps_ref, x_ref, parts_ref, w1_ref, b1_ref, w2_ref, b2_ref,
                   gamma_ref, beta_ref, lw_ref, lb_ref, o_ref,
                   h2_scr, s_scr, q_scr):
    p = pl.program_id(0)
    i = pl.program_id(1)

    @pl.when(p == 0)
    def _():
        agg = parts_ref[0] + parts_ref[1]
        h0 = x_ref[...] * (1.0 + eps_ref[0]) + agg
        h1 = jnp.maximum(
            jnp.dot(h0, w1_ref[...], preferred_element_type=jnp.float32)
            + b1_ref[...], 0.0)
        h2 = jnp.dot(h1.astype(jnp.bfloat16),
                     w2_ref[...].astype(jnp.bfloat16),
                     preferred_element_type=jnp.float32) + b2_ref[...]
        h2_scr[pl.ds(i * _BLK, _BLK), :] = h2

        @pl.when(i == 0)
        def _():
            s_scr[...] = jnp.zeros_like(s_scr)
            q_scr[...] = jnp.zeros_like(q_scr)

        s_scr[...] += jnp.sum(h2, axis=0, keepdims=True)
        q_scr[...] += jnp.sum(h2 * h2, axis=0, keepdims=True)

    @pl.when(p == 1)
    def _():
        mean = s_scr[...] * (1.0 / N)
        var = q_scr[...] * (1.0 / N) - mean * mean
        inv = lax.rsqrt(var + 1e-5)
        scale = gamma_ref[...] * inv
        shift = beta_ref[...] - mean * scale
        h2 = h2_scr[pl.ds(i * _BLK, _BLK), :]
        h3 = jnp.maximum(h2 * scale + shift, 0.0)
        o_ref[...] = (jnp.dot(h3.astype(jnp.bfloat16),
                              lw_ref[...].astype(jnp.bfloat16),
                              preferred_element_type=jnp.float32)
                      + lb_ref[...])


def kernel(x, edge_index, eps, W1, b1, W2, b2, gamma, beta, lin_W, lin_b):
    ei = edge_index.reshape(2, _NUM_CHUNKS, _CHUNK)
    parts = _sc_agg(x, ei)

    eps1 = jnp.reshape(eps, (1,))
    const = lambda p, i: (0, 0)
    out = pl.pallas_call(
        _fused_tc_body,
        grid=(2, _G),
        in_specs=[
            pl.BlockSpec(memory_space=pltpu.SMEM),
            pl.BlockSpec((_BLK, F), lambda p, i: (i * (1 - p), 0)),
            pl.BlockSpec((_NC, _BLK, F), lambda p, i: (0, i * (1 - p), 0)),
            pl.BlockSpec((F, H), const),
            pl.BlockSpec((1, H), const),
            pl.BlockSpec((H, H), const),
            pl.BlockSpec((1, H), const),
            pl.BlockSpec((1, H), const),
            pl.BlockSpec((1, H), const),
            pl.BlockSpec((H, H), const),
            pl.BlockSpec((1, H), const),
        ],
        out_specs=pl.BlockSpec((_BLK, H), lambda p, i: (i * p, 0)),
        out_shape=jax.ShapeDtypeStruct((N, H), jnp.float32),
        scratch_shapes=[
            pltpu.VMEM((N, H), jnp.float32),
            pltpu.VMEM((1, H), jnp.float32),
            pltpu.VMEM((1, H), jnp.float32),
        ],
    )(eps1, x, parts, W1, b1.reshape(1, H), W2, b2.reshape(1, H),
      gamma.reshape(1, H), beta.reshape(1, H), lin_W, lin_b.reshape(1, H))
    return out
